# baseline (speedup 1.0000x reference)
import jax
import jax.numpy as jnp
from jax import lax
from jax.experimental import pallas as pl
from jax.experimental.pallas import tpu as pltpu
from jax.experimental.pallas import tpu_sc as plsc

NUM_K = 10
FRAME_T = 8
SEG = 196
NH = 12
P = 1568
HD = 64
CH = 28
NSUP = 14
HALF = 784
NW = 32
UPW = 3
WSEG = 2 * SEG


def _sel_topk(buf, r, seg):
    iota = lax.iota(jnp.int32, 16)
    base = seg * SEG

    def leaf(j):
        off = j * 16 if j < 12 else 180
        k = buf[r, pl.ds(base + off, 16)]
        idx = iota + off
        if j == 12:
            k = jnp.where(iota >= 12, k, jnp.float32(-3.4e38))
        return k, idx

    def build(lo, hi, desc):
        if lo == hi:
            k, idx = leaf(lo)
            return plsc.sort_key_val(k, idx, descending=desc)
        mid = (lo + hi) // 2
        ka, ia = build(lo, mid, desc)
        kb, ib = build(mid + 1, hi, not desc)
        m = ka >= kb
        nk = jnp.where(m, ka, kb)
        ni = jnp.where(m, ia, ib)
        return plsc.sort_key_val(nk, ni, descending=desc)

    return build(0, 12, True)


def _bcast_lane(x, i):
    return lax.gather(
        x, jnp.full((16, 1), i, jnp.int32),
        lax.GatherDimensionNumbers(offset_dims=(), collapsed_slice_dims=(0,),
                                   start_index_map=(0,)),
        (1,), mode=lax.GatherScatterMode.PROMISE_IN_BOUNDS)


def _row_out(buf, r, seg, vref):
    ck, ci = _sel_topk(buf, r, seg)
    iota = lax.iota(jnp.int32, 16)
    mx = _bcast_lane(ck, 0)
    w = jnp.where(iota < NUM_K, jnp.exp(ck - mx), jnp.float32(0.0))
    swv = jnp.broadcast_to(jnp.sum(w), (16,))
    inv = jnp.ones((16,), jnp.float32) / swv
    accs = [jnp.zeros((16,), jnp.float32) for _ in range(4)]
    for i in range(NUM_K):
        di = ci[i]
        wvi = _bcast_lane(w, i)
        for c in range(4):
            accs[c] = accs[c] + wvi * vref[di, pl.ds(c * 16, 16)]
    return [a * inv for a in accs]


def _body(as_hbm, at_hbm, vs_hbm, vt_hbm, out_hbm,
          bs0, bs1, bt0, bt1, vs0, vs1, vt0, vt1,
          sqv, sem_s0, sem_s1, sem_t0, sem_t1):
    wid = lax.axis_index("s") * 2 + lax.axis_index("c")

    def unit_body(u, sq):
        unit = wid * UPW + u
        h = unit // 8
        rem = unit % 8
        tp = rem // 2
        p0 = (rem % 2) * HALF
        col0 = tp * WSEG

        pltpu.sync_copy(vs_hbm.at[h, 2 * tp], vs0)
        pltpu.sync_copy(vs_hbm.at[h, 2 * tp + 1], vs1)
        pltpu.sync_copy(vt_hbm.at[h, 2 * tp], vt0)
        pltpu.sync_copy(vt_hbm.at[h, 2 * tp + 1], vt1)

        def mk(chunk, hbm, buf, sem):
            src = hbm.at[h, pl.ds(p0 + chunk * CH, CH), pl.ds(col0, WSEG)]
            return pltpu.make_async_copy(src, buf, sem)

        def chunk_compute(bs, bt, sq):
            def row_body(r, sq):
                o_s0 = _row_out(bs, r, 0, vs0)
                o_t0 = _row_out(bt, r, 0, vt0)
                o_s1 = _row_out(bs, r, 1, vs1)
                o_t1 = _row_out(bt, r, 1, vt1)
                for c in range(4):
                    d0 = o_s0[c] - o_t0[c]
                    d1 = o_s1[c] - o_t1[c]
                    sq = sq + d0 * d0 + d1 * d1
                return sq
            return lax.fori_loop(0, CH, row_body, sq)

        mk(0, as_hbm, bs0, sem_s0).start()
        mk(0, at_hbm, bt0, sem_t0).start()

        def super_body(g, sq):
            mk(2 * g + 1, as_hbm, bs1, sem_s1).start()
            mk(2 * g + 1, at_hbm, bt1, sem_t1).start()
            mk(2 * g, as_hbm, bs0, sem_s0).wait()
            mk(2 * g, at_hbm, bt0, sem_t0).wait()
            sq = chunk_compute(bs0, bt0, sq)

            @pl.when(g < NSUP - 1)
            def _():
                mk(2 * g + 2, as_hbm, bs0, sem_s0).start()
                mk(2 * g + 2, at_hbm, bt0, sem_t0).start()

            mk(2 * g + 1, as_hbm, bs1, sem_s1).wait()
            mk(2 * g + 1, at_hbm, bt1, sem_t1).wait()
            return chunk_compute(bs1, bt1, sq)

        return lax.fori_loop(0, NSUP, super_body, sq)

    sq = lax.fori_loop(0, UPW, unit_body, jnp.zeros((16,), jnp.float32))
    sqv[...] = sq
    pltpu.sync_copy(sqv, out_hbm.at[wid])


_sc_call = pl.kernel(
    _body,
    out_type=jax.ShapeDtypeStruct((NW, 16), jnp.float32),
    mesh=plsc.VectorSubcoreMesh(core_axis_name="c", subcore_axis_name="s"),
    compiler_params=pltpu.CompilerParams(
        use_tc_tiling_on_sc=False, needs_layout_passes=False),
    scratch_types=[
        pltpu.VMEM((CH, WSEG), jnp.float32),
        pltpu.VMEM((CH, WSEG), jnp.float32),
        pltpu.VMEM((CH, WSEG), jnp.float32),
        pltpu.VMEM((CH, WSEG), jnp.float32),
        pltpu.VMEM((SEG, HD), jnp.float32),
        pltpu.VMEM((SEG, HD), jnp.float32),
        pltpu.VMEM((SEG, HD), jnp.float32),
        pltpu.VMEM((SEG, HD), jnp.float32),
        pltpu.VMEM((16,), jnp.float32),
        pltpu.SemaphoreType.DMA,
        pltpu.SemaphoreType.DMA,
        pltpu.SemaphoreType.DMA,
        pltpu.SemaphoreType.DMA,
    ],
)


@jax.jit
def kernel(att_s, att_t, v_s, v_t):
    as3 = att_s.reshape(NH, P, P)
    at3 = att_t.reshape(NH, P, P)
    v_rs = v_s.reshape(NH, SEG, FRAME_T, HD).transpose(0, 2, 1, 3)
    v_rt = v_t.reshape(NH, SEG, FRAME_T, HD).transpose(0, 2, 1, 3)
    out = _sc_call(as3, at3, v_rs, v_rt)
    return jnp.sum(out) / (NH * P * FRAME_T * HD)


# speedup vs baseline: 1.7322x; 1.7322x over previous
import jax
import jax.numpy as jnp
from jax import lax
from jax.experimental import pallas as pl
from jax.experimental.pallas import tpu as pltpu
from jax.experimental.pallas import tpu_sc as plsc

NUM_K = 10
FRAME_T = 8
SEG = 196
NH = 12
P = 1568
HD = 64
CH = 28
NSUP = 14
HALF = 784
NW = 32
UPW = 3
WSEG = 2 * SEG


def _sel_topk(buf, r, seg):
    iota = lax.iota(jnp.int32, 16)
    base = seg * SEG

    def leaf(j):
        off = j * 16 if j < 12 else 180
        k = buf[r, pl.ds(base + off, 16)]
        idx = iota + off
        if j == 12:
            k = jnp.where(iota >= 12, k, jnp.float32(-3.4e38))
        return k, idx

    plans = [("leaf", j) for j in range(13)]
    while len(plans) > 1:
        nxt = [("merge", plans[i], plans[i + 1])
               for i in range(0, len(plans) - 1, 2)]
        if len(plans) % 2:
            nxt.append(plans[-1])
        plans = nxt

    dirs = {}

    def assign(node, desc):
        dirs[id(node)] = desc
        if node[0] == "merge":
            assign(node[1], desc)
            assign(node[2], not desc)

    assign(plans[0], True)

    levels = [[plans[0]]]
    while True:
        cur = levels[-1]
        child = [c for n in cur if n[0] == "merge" for c in (n[1], n[2])]
        if not child:
            break
        levels.append(child)
    cache = {}
    for lvl in reversed(levels):
        for node in lvl:
            if id(node) in cache:
                continue
            desc = dirs[id(node)]
            if node[0] == "leaf":
                k, idx = leaf(node[1])
                cache[id(node)] = plsc.sort_key_val(k, idx, descending=desc)
            else:
                ka, ia = cache[id(node[1])]
                kb, ib = cache[id(node[2])]
                m = ka >= kb
                nk = jnp.where(m, ka, kb)
                ni = jnp.where(m, ia, ib)
                cache[id(node)] = plsc.sort_key_val(nk, ni, descending=desc)
    return cache[id(plans[0])]


def _row_out(buf, r, seg, vref):
    ck, ci = _sel_topk(buf, r, seg)
    iota = lax.iota(jnp.int32, 16)
    mx = jnp.max(ck)
    w = jnp.where(iota < NUM_K, jnp.exp(ck - mx), jnp.float32(0.0))
    swv = jnp.broadcast_to(jnp.sum(w), (16,))
    inv = jnp.ones((16,), jnp.float32) / swv
    accs = [jnp.zeros((16,), jnp.float32) for _ in range(4)]
    for i in range(NUM_K):
        di = ci[i]
        wi = w[i]
        for c in range(4):
            accs[c] = accs[c] + wi * vref[di, pl.ds(c * 16, 16)]
    return [a * inv for a in accs]


def _body(as_hbm, at_hbm, vs_hbm, vt_hbm, out_hbm,
          bs0, bs1, bt0, bt1, vs0, vs1, vt0, vt1,
          sqv, sem_s0, sem_s1, sem_t0, sem_t1):
    wid = lax.axis_index("s") * 2 + lax.axis_index("c")

    def unit_body(u, sq):
        unit = wid * UPW + u
        h = unit // 8
        rem = unit % 8
        tp = rem // 2
        p0 = (rem % 2) * HALF
        col0 = tp * WSEG

        pltpu.sync_copy(vs_hbm.at[h, 2 * tp], vs0)
        pltpu.sync_copy(vs_hbm.at[h, 2 * tp + 1], vs1)
        pltpu.sync_copy(vt_hbm.at[h, 2 * tp], vt0)
        pltpu.sync_copy(vt_hbm.at[h, 2 * tp + 1], vt1)

        def mk(chunk, hbm, buf, sem):
            src = hbm.at[h, pl.ds(p0 + chunk * CH, CH), pl.ds(col0, WSEG)]
            return pltpu.make_async_copy(src, buf, sem)

        def chunk_compute(bs, bt, sq):
            def row_body(r, sq):
                o_s0 = _row_out(bs, r, 0, vs0)
                o_t0 = _row_out(bt, r, 0, vt0)
                o_s1 = _row_out(bs, r, 1, vs1)
                o_t1 = _row_out(bt, r, 1, vt1)
                for c in range(4):
                    d0 = o_s0[c] - o_t0[c]
                    d1 = o_s1[c] - o_t1[c]
                    sq = sq + d0 * d0 + d1 * d1
                return sq
            return lax.fori_loop(0, CH, row_body, sq)

        mk(0, as_hbm, bs0, sem_s0).start()
        mk(0, at_hbm, bt0, sem_t0).start()

        def super_body(g, sq):
            mk(2 * g + 1, as_hbm, bs1, sem_s1).start()
            mk(2 * g + 1, at_hbm, bt1, sem_t1).start()
            mk(2 * g, as_hbm, bs0, sem_s0).wait()
            mk(2 * g, at_hbm, bt0, sem_t0).wait()
            sq = chunk_compute(bs0, bt0, sq)

            @pl.when(g < NSUP - 1)
            def _():
                mk(2 * g + 2, as_hbm, bs0, sem_s0).start()
                mk(2 * g + 2, at_hbm, bt0, sem_t0).start()

            mk(2 * g + 1, as_hbm, bs1, sem_s1).wait()
            mk(2 * g + 1, at_hbm, bt1, sem_t1).wait()
            return chunk_compute(bs1, bt1, sq)

        return lax.fori_loop(0, NSUP, super_body, sq)

    sq = lax.fori_loop(0, UPW, unit_body, jnp.zeros((16,), jnp.float32))
    sqv[...] = sq
    pltpu.sync_copy(sqv, out_hbm.at[wid])


_sc_call = pl.kernel(
    _body,
    out_type=jax.ShapeDtypeStruct((NW, 16), jnp.float32),
    mesh=plsc.VectorSubcoreMesh(core_axis_name="c", subcore_axis_name="s"),
    compiler_params=pltpu.CompilerParams(
        use_tc_tiling_on_sc=False, needs_layout_passes=False),
    scratch_types=[
        pltpu.VMEM((CH, WSEG), jnp.float32),
        pltpu.VMEM((CH, WSEG), jnp.float32),
        pltpu.VMEM((CH, WSEG), jnp.float32),
        pltpu.VMEM((CH, WSEG), jnp.float32),
        pltpu.VMEM((SEG, HD), jnp.float32),
        pltpu.VMEM((SEG, HD), jnp.float32),
        pltpu.VMEM((SEG, HD), jnp.float32),
        pltpu.VMEM((SEG, HD), jnp.float32),
        pltpu.VMEM((16,), jnp.float32),
        pltpu.SemaphoreType.DMA,
        pltpu.SemaphoreType.DMA,
        pltpu.SemaphoreType.DMA,
        pltpu.SemaphoreType.DMA,
    ],
)


@jax.jit
def kernel(att_s, att_t, v_s, v_t):
    as3 = att_s.reshape(NH, P, P)
    at3 = att_t.reshape(NH, P, P)
    v_rs = v_s.reshape(NH, SEG, FRAME_T, HD).transpose(0, 2, 1, 3)
    v_rt = v_t.reshape(NH, SEG, FRAME_T, HD).transpose(0, 2, 1, 3)
    out = _sc_call(as3, at3, v_rs, v_rt)
    return jnp.sum(out) / (NH * P * FRAME_T * HD)


# speedup vs baseline: 1.8449x; 1.0651x over previous
import jax
import jax.numpy as jnp
from jax import lax
from jax.experimental import pallas as pl
from jax.experimental.pallas import tpu as pltpu
from jax.experimental.pallas import tpu_sc as plsc

NUM_K = 10
FRAME_T = 8
SEG = 196
NH = 12
P = 1568
HD = 64
CH = 28
NSUP = 14
HALF = 784
NW = 32
UPW = 3
WSEG = 2 * SEG


def _tree_plan():
    plans = [("leaf", j) for j in range(13)]
    while len(plans) > 1:
        nxt = [("merge", plans[i], plans[i + 1])
               for i in range(0, len(plans) - 1, 2)]
        if len(plans) % 2:
            nxt.append(plans[-1])
        plans = nxt
    root = plans[0]
    dirs = {}

    def assign(node, desc):
        dirs[id(node)] = desc
        if node[0] == "merge":
            assign(node[1], desc)
            assign(node[2], not desc)

    assign(root, True)
    levels = [[root]]
    while True:
        child = [c for n in levels[-1] if n[0] == "merge"
                 for c in (n[1], n[2])]
        if not child:
            break
        levels.append(child)
    return root, dirs, levels


_ROOT, _DIRS, _LEVELS = _tree_plan()


def _row_out_multi(specs):
    iota = lax.iota(jnp.int32, 16)

    def leaf(buf, r, seg, j):
        off = j * 16 if j < 12 else 180
        k = buf[r, pl.ds(seg * SEG + off, 16)]
        idx = iota + off
        if j == 12:
            k = jnp.where(iota >= 12, k, jnp.float32(-3.4e38))
        return k, idx

    caches = [dict() for _ in specs]
    for lvl in reversed(_LEVELS):
        for node in lvl:
            for cache, (buf, r, seg, _) in zip(caches, specs):
                if id(node) in cache:
                    continue
                desc = _DIRS[id(node)]
                if node[0] == "leaf":
                    k, idx = leaf(buf, r, seg, node[1])
                else:
                    ka, ia = cache[id(node[1])]
                    kb, ib = cache[id(node[2])]
                    m = ka >= kb
                    k = jnp.where(m, ka, kb)
                    idx = jnp.where(m, ia, ib)
                cache[id(node)] = plsc.sort_key_val(k, idx, descending=desc)

    sels = [cache[id(_ROOT)] for cache in caches]
    ws = []
    invs = []
    for ck, _ in sels:
        mx = jnp.max(ck)
        w = jnp.where(iota < NUM_K, jnp.exp(ck - mx), jnp.float32(0.0))
        ws.append(w)
    for w in ws:
        swv = jnp.broadcast_to(jnp.sum(w), (16,))
        invs.append(jnp.ones((16,), jnp.float32) / swv)
    accs = [[jnp.zeros((16,), jnp.float32) for _ in range(4)]
            for _ in specs]
    for i in range(NUM_K):
        for u, (spec, (_, ci), w) in enumerate(zip(specs, sels, ws)):
            vref = spec[3]
            di = ci[i]
            wi = w[i]
            for c in range(4):
                accs[u][c] = accs[u][c] + wi * vref[di, pl.ds(c * 16, 16)]
    return [[a * inv for a in acc] for acc, inv in zip(accs, invs)]


def _body(as_hbm, at_hbm, vs_hbm, vt_hbm, out_hbm,
          bs0, bs1, bt0, bt1, vs0, vs1, vt0, vt1,
          sqv, sem_s0, sem_s1, sem_t0, sem_t1):
    wid = lax.axis_index("s") * 2 + lax.axis_index("c")

    def unit_body(u, sq):
        unit = wid * UPW + u
        h = unit // 8
        rem = unit % 8
        tp = rem // 2
        p0 = (rem % 2) * HALF
        col0 = tp * WSEG

        pltpu.sync_copy(vs_hbm.at[h, 2 * tp], vs0)
        pltpu.sync_copy(vs_hbm.at[h, 2 * tp + 1], vs1)
        pltpu.sync_copy(vt_hbm.at[h, 2 * tp], vt0)
        pltpu.sync_copy(vt_hbm.at[h, 2 * tp + 1], vt1)

        def mk(chunk, hbm, buf, sem):
            src = hbm.at[h, pl.ds(p0 + chunk * CH, CH), pl.ds(col0, WSEG)]
            return pltpu.make_async_copy(src, buf, sem)

        def chunk_compute(bs, bt, sq):
            def row_body(r, sq):
                o_s0, o_t0, o_s1, o_t1 = _row_out_multi([
                    (bs, r, 0, vs0),
                    (bt, r, 0, vt0),
                    (bs, r, 1, vs1),
                    (bt, r, 1, vt1),
                ])
                for c in range(4):
                    d0 = o_s0[c] - o_t0[c]
                    d1 = o_s1[c] - o_t1[c]
                    sq = sq + d0 * d0 + d1 * d1
                return sq
            return lax.fori_loop(0, CH, row_body, sq)

        mk(0, as_hbm, bs0, sem_s0).start()
        mk(0, at_hbm, bt0, sem_t0).start()

        def super_body(g, sq):
            mk(2 * g + 1, as_hbm, bs1, sem_s1).start()
            mk(2 * g + 1, at_hbm, bt1, sem_t1).start()
            mk(2 * g, as_hbm, bs0, sem_s0).wait()
            mk(2 * g, at_hbm, bt0, sem_t0).wait()
            sq = chunk_compute(bs0, bt0, sq)

            @pl.when(g < NSUP - 1)
            def _():
                mk(2 * g + 2, as_hbm, bs0, sem_s0).start()
                mk(2 * g + 2, at_hbm, bt0, sem_t0).start()

            mk(2 * g + 1, as_hbm, bs1, sem_s1).wait()
            mk(2 * g + 1, at_hbm, bt1, sem_t1).wait()
            return chunk_compute(bs1, bt1, sq)

        return lax.fori_loop(0, NSUP, super_body, sq)

    sq = lax.fori_loop(0, UPW, unit_body, jnp.zeros((16,), jnp.float32))
    sqv[...] = sq
    pltpu.sync_copy(sqv, out_hbm.at[wid])


_sc_call = pl.kernel(
    _body,
    out_type=jax.ShapeDtypeStruct((NW, 16), jnp.float32),
    mesh=plsc.VectorSubcoreMesh(core_axis_name="c", subcore_axis_name="s"),
    compiler_params=pltpu.CompilerParams(
        use_tc_tiling_on_sc=False, needs_layout_passes=False),
    scratch_types=[
        pltpu.VMEM((CH, WSEG), jnp.float32),
        pltpu.VMEM((CH, WSEG), jnp.float32),
        pltpu.VMEM((CH, WSEG), jnp.float32),
        pltpu.VMEM((CH, WSEG), jnp.float32),
        pltpu.VMEM((SEG, HD), jnp.float32),
        pltpu.VMEM((SEG, HD), jnp.float32),
        pltpu.VMEM((SEG, HD), jnp.float32),
        pltpu.VMEM((SEG, HD), jnp.float32),
        pltpu.VMEM((16,), jnp.float32),
        pltpu.SemaphoreType.DMA,
        pltpu.SemaphoreType.DMA,
        pltpu.SemaphoreType.DMA,
        pltpu.SemaphoreType.DMA,
    ],
)


@jax.jit
def kernel(att_s, att_t, v_s, v_t):
    as3 = att_s.reshape(NH, P, P)
    at3 = att_t.reshape(NH, P, P)
    v_rs = v_s.reshape(NH, SEG, FRAME_T, HD).transpose(0, 2, 1, 3)
    v_rt = v_t.reshape(NH, SEG, FRAME_T, HD).transpose(0, 2, 1, 3)
    out = _sc_call(as3, at3, v_rs, v_rt)
    return jnp.sum(out) / (NH * P * FRAME_T * HD)


# speedup vs baseline: 1.8544x; 1.0051x over previous
import jax
import jax.numpy as jnp
from jax import lax
from jax.experimental import pallas as pl
from jax.experimental.pallas import tpu as pltpu
from jax.experimental.pallas import tpu_sc as plsc

NUM_K = 10
FRAME_T = 8
SEG = 196
NH = 12
P = 1568
HD = 64
CH = 49
NSUP = 8
HALF = 784
NW = 32
UPW = 3
WSEG = 2 * SEG


def _tree_plan():
    plans = [("leaf", j) for j in range(13)]
    while len(plans) > 1:
        nxt = [("merge", plans[i], plans[i + 1])
               for i in range(0, len(plans) - 1, 2)]
        if len(plans) % 2:
            nxt.append(plans[-1])
        plans = nxt
    root = plans[0]
    dirs = {}

    def assign(node, desc):
        dirs[id(node)] = desc
        if node[0] == "merge":
            assign(node[1], desc)
            assign(node[2], not desc)

    assign(root, True)
    levels = [[root]]
    while True:
        child = [c for n in levels[-1] if n[0] == "merge"
                 for c in (n[1], n[2])]
        if not child:
            break
        levels.append(child)
    return root, dirs, levels


_ROOT, _DIRS, _LEVELS = _tree_plan()


def _row_out_multi(specs):
    iota = lax.iota(jnp.int32, 16)

    def leaf(buf, r, seg, j):
        off = j * 16 if j < 12 else 180
        k = buf[r, pl.ds(seg * SEG + off, 16)]
        idx = iota + off
        if j == 12:
            k = jnp.where(iota >= 12, k, jnp.float32(-3.4e38))
        return k, idx

    caches = [dict() for _ in specs]

    def emit_level(u, lvl):
        cache = caches[u]
        buf, r, seg, _ = specs[u]
        for node in lvl:
            if id(node) in cache:
                continue
            desc = _DIRS[id(node)]
            if node[0] == "leaf":
                k, idx = leaf(buf, r, seg, node[1])
            else:
                ka, ia = cache[id(node[1])]
                kb, ib = cache[id(node[2])]
                m = ka >= kb
                k = jnp.where(m, ka, kb)
                idx = jnp.where(m, ia, ib)
            cache[id(node)] = plsc.sort_key_val(k, idx, descending=desc)

    lvls = list(reversed(_LEVELS))
    nl = len(lvls)
    ga = list(range(0, len(specs), 2))
    gb = list(range(1, len(specs), 2))
    sched = []
    for li in range(nl):
        sched.append(("a", li))
        if li >= 2:
            sched.append(("b", li - 2))
    sched.append(("b", nl - 2))
    sched.append(("b", nl - 1))
    for grp, li in sched:
        for u in (ga if grp == "a" else gb):
            emit_level(u, lvls[li])

    sels = [cache[id(_ROOT)] for cache in caches]
    ws = []
    invs = []
    for ck, _ in sels:
        mx = jnp.max(ck)
        w = jnp.where(iota < NUM_K, jnp.exp(ck - mx), jnp.float32(0.0))
        ws.append(w)
    for w in ws:
        swv = jnp.broadcast_to(jnp.sum(w), (16,))
        invs.append(jnp.ones((16,), jnp.float32) / swv)
    accs = [[jnp.zeros((16,), jnp.float32) for _ in range(4)]
            for _ in specs]
    for i in range(NUM_K):
        for u, (spec, (_, ci), w) in enumerate(zip(specs, sels, ws)):
            vref = spec[3]
            di = ci[i]
            wi = w[i]
            for c in range(2):
                g = vref[di, pl.ds(c * 32, 32)]
                ga, gb = plsc.unpack(g, format=plsc.PackFormat.INTERLEAVED)
                accs[u][2 * c] = accs[u][2 * c] + wi * ga
                accs[u][2 * c + 1] = accs[u][2 * c + 1] + wi * gb
    return [[a * inv for a in acc] for acc, inv in zip(accs, invs)]


def _body(as_hbm, at_hbm, vs_hbm, vt_hbm, out_hbm,
          bs0, bs1, bt0, bt1, vs0, vs1, vt0, vt1,
          sqv, sem_s0, sem_s1, sem_t0, sem_t1):
    wid = lax.axis_index("s") * 2 + lax.axis_index("c")

    def unit_body(u, sq):
        unit = wid * UPW + u
        h = unit // 8
        rem = unit % 8
        tp = rem // 2
        p0 = (rem % 2) * HALF
        col0 = tp * WSEG

        pltpu.sync_copy(vs_hbm.at[h, 2 * tp], vs0)
        pltpu.sync_copy(vs_hbm.at[h, 2 * tp + 1], vs1)
        pltpu.sync_copy(vt_hbm.at[h, 2 * tp], vt0)
        pltpu.sync_copy(vt_hbm.at[h, 2 * tp + 1], vt1)

        def mk(chunk, hbm, buf, sem):
            src = hbm.at[h, pl.ds(p0 + chunk * CH, CH), pl.ds(col0, WSEG)]
            return pltpu.make_async_copy(src, buf, sem)

        def chunk_compute(bs, bt, sq):
            def row_body(r, sq):
                o_s0, o_t0, o_s1, o_t1 = _row_out_multi([
                    (bs, r, 0, vs0),
                    (bt, r, 0, vt0),
                    (bs, r, 1, vs1),
                    (bt, r, 1, vt1),
                ])
                for c in range(4):
                    d0 = o_s0[c] - o_t0[c]
                    d1 = o_s1[c] - o_t1[c]
                    sq = sq + d0 * d0 + d1 * d1
                return sq
            return lax.fori_loop(0, CH, row_body, sq)

        mk(0, as_hbm, bs0, sem_s0).start()
        mk(0, at_hbm, bt0, sem_t0).start()

        def super_body(g, sq):
            mk(2 * g + 1, as_hbm, bs1, sem_s1).start()
            mk(2 * g + 1, at_hbm, bt1, sem_t1).start()
            mk(2 * g, as_hbm, bs0, sem_s0).wait()
            mk(2 * g, at_hbm, bt0, sem_t0).wait()
            sq = chunk_compute(bs0, bt0, sq)

            @pl.when(g < NSUP - 1)
            def _():
                mk(2 * g + 2, as_hbm, bs0, sem_s0).start()
                mk(2 * g + 2, at_hbm, bt0, sem_t0).start()

            mk(2 * g + 1, as_hbm, bs1, sem_s1).wait()
            mk(2 * g + 1, at_hbm, bt1, sem_t1).wait()
            return chunk_compute(bs1, bt1, sq)

        return lax.fori_loop(0, NSUP, super_body, sq)

    sq = lax.fori_loop(0, UPW, unit_body, jnp.zeros((16,), jnp.float32))
    sqv[...] = sq
    pltpu.sync_copy(sqv, out_hbm.at[wid])


_sc_call = pl.kernel(
    _body,
    out_type=jax.ShapeDtypeStruct((NW, 16), jnp.float32),
    mesh=plsc.VectorSubcoreMesh(core_axis_name="c", subcore_axis_name="s"),
    compiler_params=pltpu.CompilerParams(
        use_tc_tiling_on_sc=False, needs_layout_passes=False),
    scratch_types=[
        pltpu.VMEM((CH, WSEG), jnp.float32),
        pltpu.VMEM((CH, WSEG), jnp.float32),
        pltpu.VMEM((CH, WSEG), jnp.float32),
        pltpu.VMEM((CH, WSEG), jnp.float32),
        pltpu.VMEM((SEG, HD), jnp.bfloat16),
        pltpu.VMEM((SEG, HD), jnp.bfloat16),
        pltpu.VMEM((SEG, HD), jnp.bfloat16),
        pltpu.VMEM((SEG, HD), jnp.bfloat16),
        pltpu.VMEM((16,), jnp.float32),
        pltpu.SemaphoreType.DMA,
        pltpu.SemaphoreType.DMA,
        pltpu.SemaphoreType.DMA,
        pltpu.SemaphoreType.DMA,
    ],
)


@jax.jit
def kernel(att_s, att_t, v_s, v_t):
    as3 = att_s.reshape(NH, P, P)
    at3 = att_t.reshape(NH, P, P)
    v_rs = v_s.reshape(NH, SEG, FRAME_T, HD).transpose(0, 2, 1, 3)
    v_rt = v_t.reshape(NH, SEG, FRAME_T, HD).transpose(0, 2, 1, 3)
    v_rs = v_rs.astype(jnp.bfloat16)
    v_rt = v_rt.astype(jnp.bfloat16)
    out = _sc_call(as3, at3, v_rs, v_rt)
    return jnp.sum(out) / (NH * P * FRAME_T * HD)


# speedup vs baseline: 1.8612x; 1.0037x over previous
import jax
import jax.numpy as jnp
from jax import lax
from jax.experimental import pallas as pl
from jax.experimental.pallas import tpu as pltpu
from jax.experimental.pallas import tpu_sc as plsc

NUM_K = 10
FRAME_T = 8
SEG = 196
NH = 12
P = 1568
HD = 64
CH = 28
NSUP = 14
HALF = 784
NW = 32
UPW = 3
WSEG = 2 * SEG


def _tree_plan():
    plans = [("leaf", j) for j in range(13)]
    while len(plans) > 1:
        nxt = [("merge", plans[i], plans[i + 1])
               for i in range(0, len(plans) - 1, 2)]
        if len(plans) % 2:
            nxt.append(plans[-1])
        plans = nxt
    root = plans[0]
    dirs = {}

    def assign(node, desc):
        dirs[id(node)] = desc
        if node[0] == "merge":
            assign(node[1], desc)
            assign(node[2], not desc)

    assign(root, True)
    levels = [[root]]
    while True:
        child = [c for n in levels[-1] if n[0] == "merge"
                 for c in (n[1], n[2])]
        if not child:
            break
        levels.append(child)
    return root, dirs, levels


_ROOT, _DIRS, _LEVELS = _tree_plan()


def _row_out_multi(specs):
    iota = lax.iota(jnp.int32, 16)

    def leaf(buf, r, seg, j):
        off = j * 16 if j < 12 else 180
        k = buf[r, pl.ds(seg * SEG + off, 16)]
        idx = iota + off
        if j == 12:
            k = jnp.where(iota >= 12, k, jnp.float32(-3.4e38))
        return k, idx

    caches = [dict() for _ in specs]

    def emit_level(u, lvl):
        cache = caches[u]
        buf, r, seg, _ = specs[u]
        for node in lvl:
            if id(node) in cache:
                continue
            desc = _DIRS[id(node)]
            if node[0] == "leaf":
                k, idx = leaf(buf, r, seg, node[1])
            else:
                ka, ia = cache[id(node[1])]
                kb, ib = cache[id(node[2])]
                m = ka >= kb
                k = jnp.where(m, ka, kb)
                idx = jnp.where(m, ia, ib)
            cache[id(node)] = plsc.sort_key_val(k, idx, descending=desc)

    lvls = list(reversed(_LEVELS))
    nl = len(lvls)
    ga = list(range(0, len(specs), 2))
    gb = list(range(1, len(specs), 2))
    sched = []
    for li in range(nl):
        sched.append(("a", li))
        if li >= 2:
            sched.append(("b", li - 2))
    sched.append(("b", nl - 2))
    sched.append(("b", nl - 1))
    for grp, li in sched:
        for u in (ga if grp == "a" else gb):
            emit_level(u, lvls[li])

    sels = [cache[id(_ROOT)] for cache in caches]
    ws = []
    invs = []
    for ck, _ in sels:
        mx = jnp.max(ck)
        w = jnp.where(iota < NUM_K, jnp.exp(ck - mx), jnp.float32(0.0))
        ws.append(w)
    for w in ws:
        swv = jnp.broadcast_to(jnp.sum(w), (16,))
        invs.append(jnp.ones((16,), jnp.float32) / swv)
    accs = [[jnp.zeros((16,), jnp.float32) for _ in range(4)]
            for _ in specs]
    for i in range(NUM_K):
        for u, (spec, (_, ci), w) in enumerate(zip(specs, sels, ws)):
            vref = spec[3]
            di = ci[i]
            wi = w[i]
            for c in range(2):
                g = vref[di, pl.ds(c * 32, 32)]
                ga, gb = plsc.unpack(g, format=plsc.PackFormat.INTERLEAVED)
                accs[u][2 * c] = accs[u][2 * c] + wi * ga
                accs[u][2 * c + 1] = accs[u][2 * c + 1] + wi * gb
    return [[a * inv for a in acc] for acc, inv in zip(accs, invs)]


def _body(as_hbm, at_hbm, vs_hbm, vt_hbm, out_hbm,
          bs0, bs1, bt0, bt1, vs0, vs1, vt0, vt1,
          sqv, sem_s0, sem_s1, sem_t0, sem_t1):
    wid = lax.axis_index("s") * 2 + lax.axis_index("c")

    def unit_body(u, sq):
        unit = wid * UPW + u
        h = unit // 8
        rem = unit % 8
        tp = rem // 2
        p0 = (rem % 2) * HALF
        col0 = tp * WSEG

        pltpu.sync_copy(vs_hbm.at[h, 2 * tp], vs0)
        pltpu.sync_copy(vs_hbm.at[h, 2 * tp + 1], vs1)
        pltpu.sync_copy(vt_hbm.at[h, 2 * tp], vt0)
        pltpu.sync_copy(vt_hbm.at[h, 2 * tp + 1], vt1)

        def mk(chunk, hbm, buf, sem):
            src = hbm.at[h, pl.ds(p0 + chunk * CH, CH), pl.ds(col0, WSEG)]
            return pltpu.make_async_copy(src, buf, sem)

        def chunk_compute(bs, bt, sq):
            def row_body(r, sq):
                o_s0, o_t0, o_s1, o_t1 = _row_out_multi([
                    (bs, r, 0, vs0),
                    (bt, r, 0, vt0),
                    (bs, r, 1, vs1),
                    (bt, r, 1, vt1),
                ])
                for c in range(4):
                    d0 = o_s0[c] - o_t0[c]
                    d1 = o_s1[c] - o_t1[c]
                    sq = sq + d0 * d0 + d1 * d1
                return sq
            return lax.fori_loop(0, CH, row_body, sq)

        mk(0, as_hbm, bs0, sem_s0).start()
        mk(0, at_hbm, bt0, sem_t0).start()

        def super_body(g, sq):
            mk(2 * g + 1, as_hbm, bs1, sem_s1).start()
            mk(2 * g + 1, at_hbm, bt1, sem_t1).start()
            mk(2 * g, as_hbm, bs0, sem_s0).wait()
            mk(2 * g, at_hbm, bt0, sem_t0).wait()
            sq = chunk_compute(bs0, bt0, sq)

            @pl.when(g < NSUP - 1)
            def _():
                mk(2 * g + 2, as_hbm, bs0, sem_s0).start()
                mk(2 * g + 2, at_hbm, bt0, sem_t0).start()

            mk(2 * g + 1, as_hbm, bs1, sem_s1).wait()
            mk(2 * g + 1, at_hbm, bt1, sem_t1).wait()
            return chunk_compute(bs1, bt1, sq)

        return lax.fori_loop(0, NSUP, super_body, sq)

    sq = lax.fori_loop(0, UPW, unit_body, jnp.zeros((16,), jnp.float32))
    sqv[...] = sq
    pltpu.sync_copy(sqv, out_hbm.at[wid])


_sc_call = pl.kernel(
    _body,
    out_type=jax.ShapeDtypeStruct((NW, 16), jnp.float32),
    mesh=plsc.VectorSubcoreMesh(core_axis_name="c", subcore_axis_name="s"),
    compiler_params=pltpu.CompilerParams(
        use_tc_tiling_on_sc=False, needs_layout_passes=False),
    scratch_types=[
        pltpu.VMEM((CH, WSEG), jnp.float32),
        pltpu.VMEM((CH, WSEG), jnp.float32),
        pltpu.VMEM((CH, WSEG), jnp.float32),
        pltpu.VMEM((CH, WSEG), jnp.float32),
        pltpu.VMEM((SEG, HD), jnp.bfloat16),
        pltpu.VMEM((SEG, HD), jnp.bfloat16),
        pltpu.VMEM((SEG, HD), jnp.bfloat16),
        pltpu.VMEM((SEG, HD), jnp.bfloat16),
        pltpu.VMEM((16,), jnp.float32),
        pltpu.SemaphoreType.DMA,
        pltpu.SemaphoreType.DMA,
        pltpu.SemaphoreType.DMA,
        pltpu.SemaphoreType.DMA,
    ],
)


@jax.jit
def kernel(att_s, att_t, v_s, v_t):
    as3 = att_s.reshape(NH, P, P)
    at3 = att_t.reshape(NH, P, P)
    v_rs = v_s.reshape(NH, SEG, FRAME_T, HD).transpose(0, 2, 1, 3)
    v_rt = v_t.reshape(NH, SEG, FRAME_T, HD).transpose(0, 2, 1, 3)
    v_rs = v_rs.astype(jnp.bfloat16)
    v_rt = v_rt.astype(jnp.bfloat16)
    out = _sc_call(as3, at3, v_rs, v_rt)
    return jnp.sum(out) / (NH * P * FRAME_T * HD)
